# Initial kernel scaffold; baseline (speedup 1.0000x reference)
import jax
import jax.numpy as jnp
from jax.experimental import pallas as pl


def kernel(hidden_states, gate_w, w_gate_up, w_down):
    raise NotImplementedError("write your pallas kernel here")


# speedup vs baseline: 2.4325x; 2.4325x over previous
import functools

import jax
import jax.numpy as jnp
from jax.experimental import pallas as pl
from jax.experimental.pallas import tpu as pltpu

T, D, E, K, F = 2048, 768, 8, 2, 512
EP = 128


def _moe_body(x_ref, gw_ref, wgu_ref, wd_ref, out_ref, dw_ref):
    e = pl.program_id(0)
    lane = jax.lax.broadcasted_iota(jnp.int32, (T, EP), 1)

    @pl.when(e == 0)
    def _router():
        x = x_ref[...]
        logits = jnp.dot(x, gw_ref[...], preferred_element_type=jnp.float32)
        logits = jnp.where(lane < E, logits, jnp.float32(-1e30))
        m = jnp.max(logits, axis=1, keepdims=True)
        p = jnp.exp(logits - m)
        p = jnp.where(lane < E, p, 0.0)
        p = p / jnp.sum(p, axis=1, keepdims=True)
        m1 = jnp.max(p, axis=1, keepdims=True)
        a1 = jnp.min(jnp.where(p >= m1, lane, EP), axis=1, keepdims=True)
        p2 = jnp.where(lane == a1, jnp.float32(-1.0), p)
        m2 = jnp.max(p2, axis=1, keepdims=True)
        a2 = jnp.min(jnp.where(p2 >= m2, lane, EP), axis=1, keepdims=True)
        wsum = m1 + m2
        dw_ref[...] = (jnp.where(lane == a1, m1 / wsum, 0.0)
                       + jnp.where(lane == a2, m2 / wsum, 0.0))

    x = x_ref[...]
    gu = jnp.dot(x, wgu_ref[0], preferred_element_type=jnp.float32)
    g, u = gu[:, :F], gu[:, F:]
    h = g * jax.nn.sigmoid(g) * u
    y = jnp.dot(h, wd_ref[0], preferred_element_type=jnp.float32)
    w_e = jnp.sum(jnp.where(lane == e, dw_ref[...], 0.0), axis=1, keepdims=True)
    contrib = w_e * y

    @pl.when(e == 0)
    def _init():
        out_ref[...] = contrib

    @pl.when(e > 0)
    def _acc():
        out_ref[...] += contrib


@functools.partial(jax.jit, static_argnames=("interpret",))
def kernel(hidden_states, gate_w, w_gate_up, w_down, interpret=False):
    gw_pad = jnp.pad(gate_w, ((0, 0), (0, EP - E)))
    return pl.pallas_call(
        _moe_body,
        grid=(E,),
        in_specs=[
            pl.BlockSpec((T, D), lambda e: (0, 0)),
            pl.BlockSpec((D, EP), lambda e: (0, 0)),
            pl.BlockSpec((1, D, 2 * F), lambda e: (e, 0, 0)),
            pl.BlockSpec((1, F, D), lambda e: (e, 0, 0)),
        ],
        out_specs=pl.BlockSpec((T, D), lambda e: (0, 0)),
        out_shape=jax.ShapeDtypeStruct((T, D), jnp.float32),
        scratch_shapes=[pltpu.VMEM((T, EP), jnp.float32)],
        interpret=interpret,
    )(hidden_states, gw_pad, w_gate_up, w_down)
